# baseline (speedup 1.0000x reference)
import functools

import jax
import jax.numpy as jnp
from jax import lax
from jax.experimental import pallas as pl
from jax.experimental.pallas import tpu as pltpu
from jax.experimental.pallas import tpu_sc as plsc

DIM = 256
RES = 6
NUM_LEAD = 41
TABLE_PAD = 48
LANES = 16
CHUNK = 128
NBUF = 3
SUBREP = 1


def _mlp_body(emb_ref, w1_ref, b1_ref, w2_ref, b2_ref, out_ref):
    h = jnp.dot(emb_ref[...], w1_ref[...], preferred_element_type=jnp.float32)
    h = h + b1_ref[...]
    h = 0.5 * h * (1.0 + lax.erf(h * (2.0 ** -0.5)))
    out_ref[...] = (
        jnp.dot(h, w2_ref[...], preferred_element_type=jnp.float32) + b2_ref[...]
    )


def _mlp_table(emb_pad, W1, b1, W2, b2):
    return pl.pallas_call(
        _mlp_body,
        out_shape=jax.ShapeDtypeStruct((TABLE_PAD, DIM), jnp.float32),
    )(emb_pad, W1, b1.reshape(1, -1), W2, b2.reshape(1, -1))


@functools.lru_cache(maxsize=None)
def _make_gather(B):
    info = plsc.get_sparse_core_info()
    NC, NS = info.num_cores, info.num_subcores
    NW = NC * NS
    n_ch = B // (NW * CHUNK)
    mesh = plsc.VectorSubcoreMesh(core_axis_name="c", subcore_axis_name="s")

    @functools.partial(
        pl.kernel,
        mesh=mesh,
        out_type=(
            jax.ShapeDtypeStruct((B, DIM), jnp.float32),
            jax.ShapeDtypeStruct((NW * SUBREP * TABLE_PAD, DIM), jnp.float32),
        ),
        scratch_types=(
            [pltpu.VMEM((TABLE_PAD, DIM), jnp.float32),
             pltpu.VMEM((n_ch, CHUNK), jnp.int32)]
            + [pltpu.VMEM((CHUNK, DIM), jnp.float32) for _ in range(NBUF)]
            + [pltpu.SemaphoreType.DMA for _ in range(2 * NBUF + 1)]
        ),
    )
    def gather_k(lh_hbm, table_hbm, out_hbm, reps_hbm, tstage, idx_v,
                 *bufs_sems):
        bufs = bufs_sems[:NBUF]
        gsems = bufs_sems[NBUF:2 * NBUF]
        wsems = bufs_sems[2 * NBUF:3 * NBUF]
        rsem = bufs_sems[3 * NBUF]
        wid = lax.axis_index("s") * NC + lax.axis_index("c")
        base = wid * n_ch * CHUNK

        pltpu.sync_copy(table_hbm, tstage)
        rep_writes = []
        for r in range(SUBREP):
            rep_writes.append(pltpu.async_copy(
                tstage,
                reps_hbm.at[pl.ds((wid * SUBREP + r) * TABLE_PAD, TABLE_PAD)],
                rsem,
            ))
        for j in range(n_ch):
            pltpu.sync_copy(lh_hbm.at[pl.ds(base + j * CHUNK, CHUNK)],
                            idx_v.at[j])
        rep_pat = (lax.iota(jnp.int32, LANES) & (SUBREP - 1)) * TABLE_PAD
        for j in range(n_ch):
            for i in range(CHUNK // LANES):
                v = idx_v[j, pl.ds(i * LANES, LANES)]
                f = v.astype(jnp.float32) / float(RES)
                idx_v[j, pl.ds(i * LANES, LANES)] = (
                    jnp.clip(f.astype(jnp.int32), 0, NUM_LEAD - 1)
                    + (wid * (SUBREP * TABLE_PAD))
                    + rep_pat
                )
        for w in rep_writes:
            w.wait()

        gathers = [None] * n_ch
        writes = [None] * n_ch

        def gather(j):
            return pltpu.async_copy(
                reps_hbm.at[idx_v.at[j]], bufs[j % NBUF], gsems[j % NBUF]
            )

        for j in range(min(NBUF, n_ch)):
            gathers[j] = gather(j)
        waited = set()
        for j in range(n_ch):
            if j >= NBUF:
                writes[j - NBUF].wait()
                waited.add(j - NBUF)
                gathers[j] = gather(j)
            gathers[j].wait()
            writes[j] = pltpu.async_copy(
                bufs[j % NBUF],
                out_hbm.at[pl.ds(base + j * CHUNK, CHUNK)],
                wsems[j % NBUF],
            )
        for j in range(n_ch):
            if j not in waited:
                writes[j].wait()

    return gather_k


def kernel(lead_hours, lead_embed, W1, b1, W2, b2):
    B = lead_hours.shape[0]
    table = _mlp_table(
        jnp.pad(lead_embed, ((0, TABLE_PAD - NUM_LEAD), (0, 0))), W1, b1, W2, b2
    )
    out, _ = _make_gather(B)(lead_hours.astype(jnp.int32), table)
    return out


# speedup vs baseline: 1.0002x; 1.0002x over previous
import functools

import jax
import jax.numpy as jnp
from jax import lax
from jax.experimental import pallas as pl
from jax.experimental.pallas import tpu as pltpu
from jax.experimental.pallas import tpu_sc as plsc

DIM = 256
RES = 6
NUM_LEAD = 41
TABLE_PAD = 48
LANES = 16
CHUNK = 64
NBUF = 6
SUBREP = 2


def _mlp_body(emb_ref, w1_ref, b1_ref, w2_ref, b2_ref, out_ref):
    h = jnp.dot(emb_ref[...], w1_ref[...], preferred_element_type=jnp.float32)
    h = h + b1_ref[...]
    h = 0.5 * h * (1.0 + lax.erf(h * (2.0 ** -0.5)))
    out_ref[...] = (
        jnp.dot(h, w2_ref[...], preferred_element_type=jnp.float32) + b2_ref[...]
    )


def _mlp_table(emb_pad, W1, b1, W2, b2):
    return pl.pallas_call(
        _mlp_body,
        out_shape=jax.ShapeDtypeStruct((TABLE_PAD, DIM), jnp.float32),
    )(emb_pad, W1, b1.reshape(1, -1), W2, b2.reshape(1, -1))


@functools.lru_cache(maxsize=None)
def _make_gather(B):
    info = plsc.get_sparse_core_info()
    NC, NS = info.num_cores, info.num_subcores
    NW = NC * NS
    n_ch = B // (NW * CHUNK)
    mesh = plsc.VectorSubcoreMesh(core_axis_name="c", subcore_axis_name="s")

    @functools.partial(
        pl.kernel,
        mesh=mesh,
        out_type=(
            jax.ShapeDtypeStruct((B, DIM), jnp.float32),
            jax.ShapeDtypeStruct((NW * SUBREP * TABLE_PAD, DIM), jnp.float32),
        ),
        scratch_types=(
            [pltpu.VMEM((TABLE_PAD, DIM), jnp.float32),
             pltpu.VMEM((n_ch, CHUNK), jnp.int32)]
            + [pltpu.VMEM((CHUNK, DIM), jnp.float32) for _ in range(NBUF)]
            + [pltpu.SemaphoreType.DMA for _ in range(2 * NBUF + 1)]
        ),
    )
    def gather_k(lh_hbm, table_hbm, out_hbm, reps_hbm, tstage, idx_v,
                 *bufs_sems):
        bufs = bufs_sems[:NBUF]
        gsems = bufs_sems[NBUF:2 * NBUF]
        wsems = bufs_sems[2 * NBUF:3 * NBUF]
        rsem = bufs_sems[3 * NBUF]
        wid = lax.axis_index("s") * NC + lax.axis_index("c")
        base = wid * n_ch * CHUNK

        pltpu.sync_copy(table_hbm, tstage)
        rep_writes = []
        for r in range(SUBREP):
            rep_writes.append(pltpu.async_copy(
                tstage,
                reps_hbm.at[pl.ds((wid * SUBREP + r) * TABLE_PAD, TABLE_PAD)],
                rsem,
            ))
        for j in range(n_ch):
            pltpu.sync_copy(lh_hbm.at[pl.ds(base + j * CHUNK, CHUNK)],
                            idx_v.at[j])
        rep_pat = (lax.iota(jnp.int32, LANES) & (SUBREP - 1)) * TABLE_PAD
        for j in range(n_ch):
            for i in range(CHUNK // LANES):
                v = idx_v[j, pl.ds(i * LANES, LANES)]
                f = v.astype(jnp.float32) / float(RES)
                idx_v[j, pl.ds(i * LANES, LANES)] = (
                    jnp.clip(f.astype(jnp.int32), 0, NUM_LEAD - 1)
                    + (wid * (SUBREP * TABLE_PAD))
                    + rep_pat
                )
        for w in rep_writes:
            w.wait()

        gathers = [None] * n_ch
        writes = [None] * n_ch

        def gather(j):
            return pltpu.async_copy(
                reps_hbm.at[idx_v.at[j]], bufs[j % NBUF], gsems[j % NBUF]
            )

        for j in range(min(NBUF, n_ch)):
            gathers[j] = gather(j)
        waited = set()
        for j in range(n_ch):
            if j >= NBUF:
                writes[j - NBUF].wait()
                waited.add(j - NBUF)
                gathers[j] = gather(j)
            gathers[j].wait()
            writes[j] = pltpu.async_copy(
                bufs[j % NBUF],
                out_hbm.at[pl.ds(base + j * CHUNK, CHUNK)],
                wsems[j % NBUF],
            )
        for j in range(n_ch):
            if j not in waited:
                writes[j].wait()

    return gather_k


def kernel(lead_hours, lead_embed, W1, b1, W2, b2):
    B = lead_hours.shape[0]
    table = _mlp_table(
        jnp.pad(lead_embed, ((0, TABLE_PAD - NUM_LEAD), (0, 0))), W1, b1, W2, b2
    )
    out, _ = _make_gather(B)(lead_hours.astype(jnp.int32), table)
    return out


# speedup vs baseline: 1.0604x; 1.0602x over previous
import functools

import jax
import jax.numpy as jnp
from jax import lax
from jax.experimental import pallas as pl
from jax.experimental.pallas import tpu as pltpu
from jax.experimental.pallas import tpu_sc as plsc

DIM = 256
RES = 6
NUM_LEAD = 41
TABLE_PAD = 48
LANES = 16
CHUNK = 128
NBUF = 3
SUBREP = 2


def _mlp_body(emb_ref, w1_ref, b1_ref, w2_ref, b2_ref, out_ref):
    h = jnp.dot(emb_ref[...], w1_ref[...], preferred_element_type=jnp.float32)
    h = h + b1_ref[...]
    h = 0.5 * h * (1.0 + lax.erf(h * (2.0 ** -0.5)))
    out_ref[...] = (
        jnp.dot(h, w2_ref[...], preferred_element_type=jnp.float32) + b2_ref[...]
    )


def _mlp_table(emb_pad, W1, b1, W2, b2):
    return pl.pallas_call(
        _mlp_body,
        out_shape=jax.ShapeDtypeStruct((TABLE_PAD, DIM), jnp.float32),
    )(emb_pad, W1, b1.reshape(1, -1), W2, b2.reshape(1, -1))


@functools.lru_cache(maxsize=None)
def _make_gather(B):
    info = plsc.get_sparse_core_info()
    NC, NS = info.num_cores, info.num_subcores
    NW = NC * NS
    n_ch = B // (NW * CHUNK)
    mesh = plsc.VectorSubcoreMesh(core_axis_name="c", subcore_axis_name="s")

    @functools.partial(
        pl.kernel,
        mesh=mesh,
        out_type=(
            jax.ShapeDtypeStruct((B, DIM), jnp.float32),
            jax.ShapeDtypeStruct((NW * SUBREP * TABLE_PAD, DIM), jnp.float32),
        ),
        scratch_types=(
            [pltpu.VMEM((TABLE_PAD, DIM), jnp.float32),
             pltpu.VMEM((n_ch, CHUNK), jnp.int32)]
            + [pltpu.VMEM((CHUNK, DIM), jnp.float32) for _ in range(NBUF)]
            + [pltpu.SemaphoreType.DMA for _ in range(2 * NBUF + 2)]
        ),
    )
    def gather_k(lh_hbm, table_hbm, out_hbm, reps_hbm, tstage, idx_v,
                 *bufs_sems):
        bufs = bufs_sems[:NBUF]
        gsems = bufs_sems[NBUF:2 * NBUF]
        wsems = bufs_sems[2 * NBUF:3 * NBUF]
        rsem = bufs_sems[3 * NBUF]
        tsem = bufs_sems[3 * NBUF + 1]
        wid = lax.axis_index("s") * NC + lax.axis_index("c")
        base = wid * n_ch * CHUNK

        n_piece = TABLE_PAD // 8
        table_reads = []
        for i in range(n_piece):
            p = lax.rem(i + wid, n_piece) * 8
            table_reads.append(pltpu.async_copy(
                table_hbm.at[pl.ds(p, 8)], tstage.at[pl.ds(p, 8)], tsem,
            ))
        for j in range(n_ch):
            pltpu.sync_copy(lh_hbm.at[pl.ds(base + j * CHUNK, CHUNK)],
                            idx_v.at[j])
        rep_pat = (lax.iota(jnp.int32, LANES) & (SUBREP - 1)) * TABLE_PAD
        for j in range(n_ch):
            for i in range(CHUNK // LANES):
                v = idx_v[j, pl.ds(i * LANES, LANES)]
                f = v.astype(jnp.float32) / float(RES)
                idx_v[j, pl.ds(i * LANES, LANES)] = (
                    jnp.clip(f.astype(jnp.int32), 0, NUM_LEAD - 1)
                    + (wid * (SUBREP * TABLE_PAD))
                    + rep_pat
                )
        for t in table_reads:
            t.wait()
        rep_writes = []
        for r in range(SUBREP):
            rep_writes.append(pltpu.async_copy(
                tstage,
                reps_hbm.at[pl.ds((wid * SUBREP + r) * TABLE_PAD, TABLE_PAD)],
                rsem,
            ))
        for w in rep_writes:
            w.wait()

        gathers = [None] * n_ch
        writes = [None] * n_ch

        def gather(j):
            return pltpu.async_copy(
                reps_hbm.at[idx_v.at[j]], bufs[j % NBUF], gsems[j % NBUF]
            )

        for j in range(min(NBUF, n_ch)):
            gathers[j] = gather(j)
        waited = set()
        for j in range(n_ch):
            if j >= NBUF:
                writes[j - NBUF].wait()
                waited.add(j - NBUF)
                gathers[j] = gather(j)
            gathers[j].wait()
            writes[j] = pltpu.async_copy(
                bufs[j % NBUF],
                out_hbm.at[pl.ds(base + j * CHUNK, CHUNK)],
                wsems[j % NBUF],
            )
        for j in range(n_ch):
            if j not in waited:
                writes[j].wait()

    return gather_k


def kernel(lead_hours, lead_embed, W1, b1, W2, b2):
    B = lead_hours.shape[0]
    table = _mlp_table(
        jnp.pad(lead_embed, ((0, TABLE_PAD - NUM_LEAD), (0, 0))), W1, b1, W2, b2
    )
    out, _ = _make_gather(B)(lead_hours.astype(jnp.int32), table)
    return out
